# baseline (speedup 1.0000x reference)
import functools

import jax
import jax.numpy as jnp
from jax import lax
from jax.experimental import pallas as pl
from jax.experimental.pallas import tpu as pltpu
from jax.experimental.pallas import tpu_sc as plsc

B = 1024
D = 128
K = 128000
TK = 5
NCAND = 8
CHUNK = 2560
NCOL = CHUNK // 128
NSTEPS = K // CHUNK
NSET = 2
NLVL = 2
MASK17 = (1 << 17) - 1
QSCALE = 4095.0
MAGIC = 12582912.0
KBIAS = 12224 << 17

SC_NC = 2
SC_NS = 16
NW = SC_NC * SC_NS
PERW = (B * NCAND) // NW
SUBG = 128


def _topk_body(t_ref, queue_ref, idx_ref, tn_s, keys_s):
    i = pl.program_id(0)

    @pl.when(i == 0)
    def _init():
        t = t_ref[...]
        n = jnp.sqrt(jnp.sum(t * t, axis=1, keepdims=True))
        tn_s[...] = t / jnp.maximum(n, 1e-12) * QSCALE
        keys_s[...] = jnp.zeros((B, NSET * NLVL * 128), jnp.float32)

    chunk = queue_ref[...]
    n = jnp.sqrt(jnp.sum(chunk * chunk, axis=1, keepdims=True))
    chunkn = chunk / jnp.maximum(n, 1e-12)
    st = lax.dot_general(tn_s[...], chunkn, (((1,), (1,)), ((), ())),
                         preferred_element_type=jnp.float32)
    qb = lax.bitcast_convert_type(st + MAGIC, jnp.int32)
    col = lax.broadcasted_iota(jnp.int32, (B, CHUNK), 1) + (KBIAS + i * CHUNK)
    key = lax.bitcast_convert_type((qb << 17) + col, jnp.float32)
    r = [[keys_s[:, (s * NLVL + k) * 128:(s * NLVL + k + 1) * 128]
          for k in range(NLVL)] for s in range(NSET)]
    for c in range(NCOL):
        v = key[:, c * 128:(c + 1) * 128]
        rs = r[c % NSET]
        for k in range(NLVL):
            hi = jnp.maximum(rs[k], v)
            v = jnp.minimum(rs[k], v)
            rs[k] = hi
    for s in range(NSET):
        for k in range(NLVL):
            keys_s[:, (s * NLVL + k) * 128:(s * NLVL + k + 1) * 128] = r[s][k]

    @pl.when(i == NSTEPS - 1)
    def _extract():
        s = [[keys_s[:, (j * NLVL + k) * 128:(j * NLVL + k + 1) * 128]
              for k in range(NLVL)] for j in range(NSET)]
        cols = []
        for _ in range(NCAND):
            m = jnp.max(jnp.maximum(s[0][0], s[1][0]), axis=1, keepdims=True)
            ik = lax.bitcast_convert_type(m, jnp.int32)
            cols.append(ik & MASK17)
            for j in range(NSET):
                f = s[j][0] == m
                for k in range(NLVL - 1):
                    s[j][k] = jnp.where(f, s[j][k + 1], s[j][k])
                s[j][NLVL - 1] = jnp.where(f, 0.0, s[j][NLVL - 1])
        idx_ref[...] = jnp.concatenate(cols, axis=1)


def _topk_indices(current_target, queue):
    return pl.pallas_call(
        _topk_body,
        grid=(NSTEPS,),
        in_specs=[
            pl.BlockSpec((B, D), lambda i: (0, 0)),
            pl.BlockSpec((CHUNK, D), lambda i: (i, 0)),
        ],
        out_specs=pl.BlockSpec((B, NCAND), lambda i: (0, 0)),
        out_shape=jax.ShapeDtypeStruct((B, NCAND), jnp.int32),
        scratch_shapes=[
            pltpu.VMEM((B, D), jnp.float32),
            pltpu.VMEM((B, NSET * NLVL * 128), jnp.float32),
        ],
    )(current_target, queue)


def _sc_gather(queue, labels_queue, idx_flat):
    mesh = plsc.VectorSubcoreMesh(core_axis_name="c", subcore_axis_name="s")

    @functools.partial(
        pl.kernel,
        mesh=mesh,
        out_type=[
            jax.ShapeDtypeStruct((B * NCAND, D), jnp.float32),
            jax.ShapeDtypeStruct((B * NCAND,), jnp.int32),
        ],
        scratch_types=[
            pltpu.VMEM((PERW,), jnp.int32),
            pltpu.VMEM((PERW, D), jnp.float32),
            pltpu.VMEM((PERW,), jnp.int32),
            pltpu.SemaphoreType.DMA,
            pltpu.SemaphoreType.DMA,
        ],
    )
    def k(queue_hbm, lblq_hbm, idx_hbm, rows_out, lbl_out, idx_v, rows_v, lbl_v, sem, sem2):
        wid = lax.axis_index("s") * SC_NC + lax.axis_index("c")
        base = wid * PERW
        pltpu.sync_copy(idx_hbm.at[pl.ds(base, PERW)], idx_v)
        copies = []
        for g in range(PERW // SUBG):
            off = g * SUBG
            copies.append(pltpu.async_copy(
                queue_hbm.at[idx_v.at[pl.ds(off, SUBG)]],
                rows_v.at[pl.ds(off, SUBG)], sem))
            copies.append(pltpu.async_copy(
                lblq_hbm.at[idx_v.at[pl.ds(off, SUBG)]],
                lbl_v.at[pl.ds(off, SUBG)], sem2))
        for c in copies:
            c.wait()
        pltpu.sync_copy(rows_v, rows_out.at[pl.ds(base, PERW)])
        pltpu.sync_copy(lbl_v, lbl_out.at[pl.ds(base, PERW)])

    return k(queue, labels_queue, idx_flat)


def _finish_body(q_ref, t_ref, rows_ref, lblg_ref, labels_ref, loss_ref, pur_ref):
    q = q_ref[...]
    qn = q / jnp.maximum(jnp.sqrt(jnp.sum(q * q, axis=1, keepdims=True)), 1e-12)
    t = t_ref[...]
    tn = t / jnp.maximum(jnp.sqrt(jnp.sum(t * t, axis=1, keepdims=True)), 1e-12)
    lab = labels_ref[...]
    dts, dqs, mts = [], [], []
    for j in range(NCAND):
        g = rows_ref[pl.ds(j * B, B), :]
        rin = 1.0 / jnp.maximum(
            jnp.sqrt(jnp.sum(g * g, axis=1, keepdims=True)), 1e-12)
        dts.append(jnp.sum(tn * g, axis=1, keepdims=True) * rin)
        dqs.append(2.0 - 2.0 * (jnp.sum(qn * g, axis=1, keepdims=True) * rin))
        lj = lblg_ref[pl.ds(j * B, B), :]
        mts.append((lj == lab).astype(jnp.float32))
    simt = jnp.concatenate(dts, axis=1)
    dq = jnp.concatenate(dqs, axis=1)
    mt = jnp.concatenate(mts, axis=1)
    lane = lax.broadcasted_iota(jnp.int32, (B, NCAND), 1)
    lacc = jnp.zeros((B, 1), jnp.float32)
    macc = jnp.zeros((B, 1), jnp.float32)
    for _ in range(TK):
        p = jnp.argmax(simt, axis=1).astype(jnp.int32)
        oh = lane == p[:, None]
        lacc = lacc + jnp.sum(jnp.where(oh, dq, 0.0), axis=1, keepdims=True)
        macc = macc + jnp.sum(jnp.where(oh, mt, 0.0), axis=1, keepdims=True)
        simt = jnp.where(oh, -jnp.inf, simt)
    loss_ref[...] = (jnp.sum(lacc) / (TK * B)).reshape(1, 1)
    pur_ref[...] = (jnp.sum(macc) / (TK * B)).reshape(1, 1)


def _finish(query, current_target, rows, lblg, labels):
    return pl.pallas_call(
        _finish_body,
        out_shape=[
            jax.ShapeDtypeStruct((1, 1), jnp.float32),
            jax.ShapeDtypeStruct((1, 1), jnp.float32),
        ],
    )(query, current_target, rows, lblg, labels)


def kernel(query, current_target, labels, queue, labels_queue):
    idx = _topk_indices(current_target, queue)
    idx_flat = idx.T.reshape(-1)
    rows, lblg = _sc_gather(queue, labels_queue, idx_flat)
    loss, pur = _finish(query, current_target, rows,
                        lblg.reshape(-1, 1), labels.reshape(-1, 1))
    return (loss.reshape(()), pur.reshape(()))


# speedup vs baseline: 1.0028x; 1.0028x over previous
import functools

import jax
import jax.numpy as jnp
from jax import lax
from jax.experimental import pallas as pl
from jax.experimental.pallas import tpu as pltpu
from jax.experimental.pallas import tpu_sc as plsc

B = 1024
D = 128
K = 128000
TK = 5
NCAND = 8
CHUNK = 5120
NCOL = CHUNK // 128
NSTEPS = K // CHUNK
NSET = 2
NLVL = 2
MASK17 = (1 << 17) - 1
QSCALE = 4095.0
MAGIC = 12582912.0
KBIAS = 12224 << 17

SC_NC = 2
SC_NS = 16
NW = SC_NC * SC_NS
PERW = (B * NCAND) // NW
SUBG = 128


def _topk_body(t_ref, queue_ref, idx_ref, tn_s, keys_s):
    i = pl.program_id(0)

    @pl.when(i == 0)
    def _init():
        t = t_ref[...]
        n = jnp.sqrt(jnp.sum(t * t, axis=1, keepdims=True))
        tn_s[...] = t / jnp.maximum(n, 1e-12) * QSCALE
        keys_s[...] = jnp.zeros((B, NSET * NLVL * 128), jnp.float32)

    chunk = queue_ref[...]
    n = jnp.sqrt(jnp.sum(chunk * chunk, axis=1, keepdims=True))
    chunkn = chunk / jnp.maximum(n, 1e-12)
    st = lax.dot_general(tn_s[...], chunkn, (((1,), (1,)), ((), ())),
                         preferred_element_type=jnp.float32)
    qb = lax.bitcast_convert_type(st + MAGIC, jnp.int32)
    col = lax.broadcasted_iota(jnp.int32, (B, CHUNK), 1) + (KBIAS + i * CHUNK)
    key = lax.bitcast_convert_type((qb << 17) + col, jnp.float32)
    r = [[keys_s[:, (s * NLVL + k) * 128:(s * NLVL + k + 1) * 128]
          for k in range(NLVL)] for s in range(NSET)]
    for c in range(NCOL):
        v = key[:, c * 128:(c + 1) * 128]
        rs = r[c % NSET]
        for k in range(NLVL):
            hi = jnp.maximum(rs[k], v)
            v = jnp.minimum(rs[k], v)
            rs[k] = hi
    for s in range(NSET):
        for k in range(NLVL):
            keys_s[:, (s * NLVL + k) * 128:(s * NLVL + k + 1) * 128] = r[s][k]

    @pl.when(i == NSTEPS - 1)
    def _extract():
        s = [[keys_s[:, (j * NLVL + k) * 128:(j * NLVL + k + 1) * 128]
              for k in range(NLVL)] for j in range(NSET)]
        cols = []
        for _ in range(NCAND):
            m = jnp.max(jnp.maximum(s[0][0], s[1][0]), axis=1, keepdims=True)
            ik = lax.bitcast_convert_type(m, jnp.int32)
            cols.append(ik & MASK17)
            for j in range(NSET):
                f = s[j][0] == m
                for k in range(NLVL - 1):
                    s[j][k] = jnp.where(f, s[j][k + 1], s[j][k])
                s[j][NLVL - 1] = jnp.where(f, 0.0, s[j][NLVL - 1])
        idx_ref[...] = jnp.concatenate(cols, axis=1)


def _topk_indices(current_target, queue):
    return pl.pallas_call(
        _topk_body,
        grid=(NSTEPS,),
        in_specs=[
            pl.BlockSpec((B, D), lambda i: (0, 0)),
            pl.BlockSpec((CHUNK, D), lambda i: (i, 0)),
        ],
        out_specs=pl.BlockSpec((B, NCAND), lambda i: (0, 0)),
        out_shape=jax.ShapeDtypeStruct((B, NCAND), jnp.int32),
        scratch_shapes=[
            pltpu.VMEM((B, D), jnp.float32),
            pltpu.VMEM((B, NSET * NLVL * 128), jnp.float32),
        ],
    )(current_target, queue)


def _sc_gather(queue, labels_queue, idx_flat):
    mesh = plsc.VectorSubcoreMesh(core_axis_name="c", subcore_axis_name="s")

    @functools.partial(
        pl.kernel,
        mesh=mesh,
        out_type=[
            jax.ShapeDtypeStruct((B * NCAND, D), jnp.float32),
            jax.ShapeDtypeStruct((B * NCAND,), jnp.int32),
        ],
        scratch_types=[
            pltpu.VMEM((PERW,), jnp.int32),
            pltpu.VMEM((PERW, D), jnp.float32),
            pltpu.VMEM((PERW,), jnp.int32),
            pltpu.SemaphoreType.DMA,
            pltpu.SemaphoreType.DMA,
        ],
    )
    def k(queue_hbm, lblq_hbm, idx_hbm, rows_out, lbl_out, idx_v, rows_v, lbl_v, sem, sem2):
        wid = lax.axis_index("s") * SC_NC + lax.axis_index("c")
        base = wid * PERW
        pltpu.sync_copy(idx_hbm.at[pl.ds(base, PERW)], idx_v)
        copies = []
        for g in range(PERW // SUBG):
            off = g * SUBG
            copies.append(pltpu.async_copy(
                queue_hbm.at[idx_v.at[pl.ds(off, SUBG)]],
                rows_v.at[pl.ds(off, SUBG)], sem))
            copies.append(pltpu.async_copy(
                lblq_hbm.at[idx_v.at[pl.ds(off, SUBG)]],
                lbl_v.at[pl.ds(off, SUBG)], sem2))
        for c in copies:
            c.wait()
        pltpu.sync_copy(rows_v, rows_out.at[pl.ds(base, PERW)])
        pltpu.sync_copy(lbl_v, lbl_out.at[pl.ds(base, PERW)])

    return k(queue, labels_queue, idx_flat)


def _finish_body(q_ref, t_ref, rows_ref, lblg_ref, labels_ref, loss_ref, pur_ref):
    q = q_ref[...]
    qn = q / jnp.maximum(jnp.sqrt(jnp.sum(q * q, axis=1, keepdims=True)), 1e-12)
    t = t_ref[...]
    tn = t / jnp.maximum(jnp.sqrt(jnp.sum(t * t, axis=1, keepdims=True)), 1e-12)
    lab = labels_ref[...]
    dts, dqs, mts = [], [], []
    for j in range(NCAND):
        g = rows_ref[pl.ds(j * B, B), :]
        rin = 1.0 / jnp.maximum(
            jnp.sqrt(jnp.sum(g * g, axis=1, keepdims=True)), 1e-12)
        dts.append(jnp.sum(tn * g, axis=1, keepdims=True) * rin)
        dqs.append(2.0 - 2.0 * (jnp.sum(qn * g, axis=1, keepdims=True) * rin))
        lj = lblg_ref[pl.ds(j * B, B), :]
        mts.append((lj == lab).astype(jnp.float32))
    simt = jnp.concatenate(dts, axis=1)
    dq = jnp.concatenate(dqs, axis=1)
    mt = jnp.concatenate(mts, axis=1)
    lane = lax.broadcasted_iota(jnp.int32, (B, NCAND), 1)
    lacc = jnp.zeros((B, 1), jnp.float32)
    macc = jnp.zeros((B, 1), jnp.float32)
    for _ in range(TK):
        p = jnp.argmax(simt, axis=1).astype(jnp.int32)
        oh = lane == p[:, None]
        lacc = lacc + jnp.sum(jnp.where(oh, dq, 0.0), axis=1, keepdims=True)
        macc = macc + jnp.sum(jnp.where(oh, mt, 0.0), axis=1, keepdims=True)
        simt = jnp.where(oh, -jnp.inf, simt)
    loss_ref[...] = (jnp.sum(lacc) / (TK * B)).reshape(1, 1)
    pur_ref[...] = (jnp.sum(macc) / (TK * B)).reshape(1, 1)


def _finish(query, current_target, rows, lblg, labels):
    return pl.pallas_call(
        _finish_body,
        out_shape=[
            jax.ShapeDtypeStruct((1, 1), jnp.float32),
            jax.ShapeDtypeStruct((1, 1), jnp.float32),
        ],
    )(query, current_target, rows, lblg, labels)


def kernel(query, current_target, labels, queue, labels_queue):
    idx = _topk_indices(current_target, queue)
    idx_flat = idx.T.reshape(-1)
    rows, lblg = _sc_gather(queue, labels_queue, idx_flat)
    loss, pur = _finish(query, current_target, rows,
                        lblg.reshape(-1, 1), labels.reshape(-1, 1))
    return (loss.reshape(()), pur.reshape(()))
